# Initial kernel scaffold; baseline (speedup 1.0000x reference)
import jax
import jax.numpy as jnp
from jax.experimental import pallas as pl


def kernel(x, adj, index, W, b):
    raise NotImplementedError("write your pallas kernel here")


# speedup vs baseline: 211.0120x; 211.0120x over previous
import jax
import jax.numpy as jnp
from jax import lax
from jax.experimental import pallas as pl
from jax.experimental.pallas import tpu as pltpu
from jax.experimental.pallas import tpu_sc as plsc

N = 10000
K = 2500
D = 128

CHUNK = 80
NCHUNK = N // CHUNK
NSUB = 16
SPLIT = 1248
ACC_ROWS = 1280
TRASH = ACC_ROWS - 1
TAIL0 = SPLIT - 15 * CHUNK
TAIL1 = K - SPLIT - 15 * CHUNK


def _body(x_hbm, idx_hbm, out_hbm, idx_v, xbuf, zbuf, obuf, acc):
    c = lax.axis_index("c")
    s = lax.axis_index("s")

    def _zrow(i, carry):
        for j in range(D // 16):
            zbuf[i, pl.ds(16 * j, 16)] = jnp.zeros((16,), jnp.float32)
        return carry

    lax.fori_loop(0, CHUNK, _zrow, 0)
    pltpu.sync_copy(zbuf, acc.at[pl.ds(s * CHUNK, CHUNK)])
    plsc.subcore_barrier()

    row0 = c * SPLIT
    nrows = SPLIT + (K - 2 * SPLIT) * c
    for t in range(NCHUNK // NSUB + 1):
        j = s + NSUB * t

        @pl.when(j < NCHUNK)
        def _():
            base = j * CHUNK
            pltpu.sync_copy(idx_hbm.at[pl.ds(base, CHUNK)], idx_v)
            pltpu.sync_copy(x_hbm.at[pl.ds(base, CHUNK)], xbuf)
            for q in range(CHUNK // 16):
                v = idx_v[pl.ds(16 * q, 16)] - row0
                ok = (v >= 0) & (v < nrows)
                idx_v[pl.ds(16 * q, 16)] = jnp.where(ok, v, TRASH)
            pltpu.sync_copy(xbuf, acc.at[idx_v], add=True)

    plsc.subcore_barrier()

    @pl.when(s < 15)
    def _():
        pltpu.sync_copy(acc.at[pl.ds(s * CHUNK, CHUNK)], obuf)
        pltpu.sync_copy(obuf, out_hbm.at[pl.ds(row0 + s * CHUNK, CHUNK)])

    @pl.when((s == 15) & (c == 0))
    def _():
        pltpu.sync_copy(acc.at[pl.ds(15 * CHUNK, TAIL0)],
                        obuf.at[pl.ds(0, TAIL0)])
        pltpu.sync_copy(obuf.at[pl.ds(0, TAIL0)],
                        out_hbm.at[pl.ds(15 * CHUNK, TAIL0)])

    @pl.when((s == 15) & (c == 1))
    def _():
        pltpu.sync_copy(acc.at[pl.ds(15 * CHUNK, TAIL1)],
                        obuf.at[pl.ds(0, TAIL1)])
        pltpu.sync_copy(obuf.at[pl.ds(0, TAIL1)],
                        out_hbm.at[pl.ds(SPLIT + 15 * CHUNK, TAIL1)])


@jax.jit
def _segment_sum_sc(x, index):
    mesh = plsc.VectorSubcoreMesh(core_axis_name="c", subcore_axis_name="s")
    f = pl.kernel(
        _body,
        out_type=jax.ShapeDtypeStruct((K, D), jnp.float32),
        mesh=mesh,
        scratch_types=[
            pltpu.VMEM((CHUNK,), jnp.int32),
            pltpu.VMEM((CHUNK, D), jnp.float32),
            pltpu.VMEM((CHUNK, D), jnp.float32),
            pltpu.VMEM((CHUNK, D), jnp.float32),
            pltpu.VMEM_SHARED((ACC_ROWS, D), jnp.float32),
        ],
    )
    return f(x, index)


def kernel(x, adj, index, W, b):
    del adj, W, b
    return _segment_sum_sc(x, index)


# speedup vs baseline: 256.1169x; 1.2138x over previous
import jax
import jax.numpy as jnp
from jax import lax
from jax.experimental import pallas as pl
from jax.experimental.pallas import tpu as pltpu
from jax.experimental.pallas import tpu_sc as plsc

N = 10000
K = 2500
D = 128

CHUNK = 128
NFULL = N // CHUNK
TAIL = N - NFULL * CHUNK
NSUB = 16
STEPS = 5
SPLIT = 1248
ACC_ROWS = 1280
TRASH = ACC_ROWS - 1
ZROWS = ACC_ROWS // NSUB
TAIL0 = SPLIT - 15 * ZROWS
TAIL1 = K - SPLIT - 15 * ZROWS


def _body(x_hbm, idx_hbm, out_hbm, idx2, xbuf, zbuf, acc, sem_i, sem_x):
    c = lax.axis_index("c")
    s = lax.axis_index("s")

    for t in range(STEPS):
        j = s + NSUB * t

        @pl.when(j < NFULL)
        def _():
            pltpu.async_copy(idx_hbm.at[pl.ds(j * CHUNK, CHUNK)],
                             idx2.at[t], sem_i)
            pltpu.async_copy(x_hbm.at[pl.ds(j * CHUNK, CHUNK)],
                             xbuf.at[t], sem_x)

        @pl.when(j == NFULL + 1)
        def _():
            pltpu.async_copy(idx_hbm.at[pl.ds(NFULL * CHUNK, TAIL)],
                             idx2.at[STEPS - 1, pl.ds(0, TAIL)], sem_i)
            pltpu.async_copy(x_hbm.at[pl.ds(NFULL * CHUNK, TAIL)],
                             xbuf.at[STEPS - 1, pl.ds(0, TAIL)], sem_x)

    def _zrow(i, carry):
        for g in range(D // 16):
            zbuf[i, pl.ds(16 * g, 16)] = jnp.zeros((16,), jnp.float32)
        return carry

    lax.fori_loop(0, ZROWS, _zrow, 0)
    pltpu.sync_copy(zbuf, acc.at[pl.ds(s * ZROWS, ZROWS)])
    plsc.subcore_barrier()

    for t in range(STEPS):
        j = s + NSUB * t

        @pl.when(j < NFULL)
        def _():
            pltpu.make_async_copy(idx_hbm.at[pl.ds(j * CHUNK, CHUNK)],
                                  idx2.at[t], sem_i).wait()
            pltpu.make_async_copy(x_hbm.at[pl.ds(j * CHUNK, CHUNK)],
                                  xbuf.at[t], sem_x).wait()

        @pl.when(j == NFULL + 1)
        def _():
            pltpu.make_async_copy(idx_hbm.at[pl.ds(NFULL * CHUNK, TAIL)],
                                  idx2.at[STEPS - 1, pl.ds(0, TAIL)],
                                  sem_i).wait()
            pltpu.make_async_copy(x_hbm.at[pl.ds(NFULL * CHUNK, TAIL)],
                                  xbuf.at[STEPS - 1, pl.ds(0, TAIL)],
                                  sem_x).wait()

    row0 = c * SPLIT
    nrows = SPLIT + (K - 2 * SPLIT) * c
    for t in range(STEPS):
        j = s + NSUB * t

        @pl.when(j < NFULL)
        def _():
            for q in range(CHUNK // 16):
                v = idx2[t, pl.ds(16 * q, 16)] - row0
                ok = (v >= 0) & (v < nrows)
                idx2[t, pl.ds(16 * q, 16)] = jnp.where(ok, v, TRASH)
            pltpu.sync_copy(xbuf.at[t], acc.at[idx2.at[t]], add=True)

        @pl.when(j == NFULL + 1)
        def _():
            v = idx2[STEPS - 1, pl.ds(0, TAIL)] - row0
            ok = (v >= 0) & (v < nrows)
            idx2[STEPS - 1, pl.ds(0, TAIL)] = jnp.where(ok, v, TRASH)
            for q in range(1, CHUNK // 16):
                idx2[STEPS - 1, pl.ds(16 * q, 16)] = jnp.full(
                    (16,), TRASH, jnp.int32)
            pltpu.sync_copy(xbuf.at[STEPS - 1],
                            acc.at[idx2.at[STEPS - 1]], add=True)

    plsc.subcore_barrier()

    @pl.when(s < 15)
    def _():
        pltpu.sync_copy(acc.at[pl.ds(s * ZROWS, ZROWS)], zbuf)
        pltpu.sync_copy(zbuf, out_hbm.at[pl.ds(row0 + s * ZROWS, ZROWS)])

    @pl.when((s == 15) & (c == 0))
    def _():
        pltpu.sync_copy(acc.at[pl.ds(15 * ZROWS, TAIL0)],
                        zbuf.at[pl.ds(0, TAIL0)])
        pltpu.sync_copy(zbuf.at[pl.ds(0, TAIL0)],
                        out_hbm.at[pl.ds(15 * ZROWS, TAIL0)])

    @pl.when((s == 15) & (c == 1))
    def _():
        pltpu.sync_copy(acc.at[pl.ds(15 * ZROWS, TAIL1)],
                        zbuf.at[pl.ds(0, TAIL1)])
        pltpu.sync_copy(zbuf.at[pl.ds(0, TAIL1)],
                        out_hbm.at[pl.ds(SPLIT + 15 * ZROWS, TAIL1)])


@jax.jit
def _segment_sum_sc(x, index):
    mesh = plsc.VectorSubcoreMesh(core_axis_name="c", subcore_axis_name="s")
    f = pl.kernel(
        _body,
        out_type=jax.ShapeDtypeStruct((K, D), jnp.float32),
        mesh=mesh,
        scratch_types=[
            pltpu.VMEM((STEPS, CHUNK), jnp.int32),
            pltpu.VMEM((STEPS, CHUNK, D), jnp.float32),
            pltpu.VMEM((ZROWS, D), jnp.float32),
            pltpu.VMEM_SHARED((ACC_ROWS, D), jnp.float32),
            pltpu.SemaphoreType.DMA,
            pltpu.SemaphoreType.DMA,
        ],
    )
    return f(x, index)


def kernel(x, adj, index, W, b):
    del adj, W, b
    return _segment_sum_sc(x, index)


# speedup vs baseline: 289.8979x; 1.1319x over previous
import jax
import jax.numpy as jnp
from jax import lax
from jax.experimental import pallas as pl
from jax.experimental.pallas import tpu as pltpu
from jax.experimental.pallas import tpu_sc as plsc

N = 10000
K = 2500
D = 128

CHUNK = 128
NFULL = N // CHUNK
TAIL = N - NFULL * CHUNK
NSUB = 16
STEPS = 5
ACC_ROWS = 2560
TRASH = ACC_ROWS - 1
ZROWS = ACC_ROWS // NSUB
OTAIL = K - 15 * ZROWS


def _body(x_hbm, idx_hbm, out_hbm, idx2, xbuf, zbuf, acc, sem_i, sem_x):
    s = lax.axis_index("s")

    for t in range(STEPS):
        j = s + NSUB * t

        @pl.when(j < NFULL)
        def _():
            pltpu.async_copy(idx_hbm.at[pl.ds(j * CHUNK, CHUNK)],
                             idx2.at[t], sem_i)
            pltpu.async_copy(x_hbm.at[pl.ds(j * CHUNK, CHUNK)],
                             xbuf.at[t], sem_x)

        @pl.when(j == NFULL + 1)
        def _():
            pltpu.async_copy(idx_hbm.at[pl.ds(NFULL * CHUNK, TAIL)],
                             idx2.at[STEPS - 1, pl.ds(0, TAIL)], sem_i)
            pltpu.async_copy(x_hbm.at[pl.ds(NFULL * CHUNK, TAIL)],
                             xbuf.at[STEPS - 1, pl.ds(0, TAIL)], sem_x)

    def _zrow(i, carry):
        for g in range(D // 16):
            zbuf[i, pl.ds(16 * g, 16)] = jnp.zeros((16,), jnp.float32)
        return carry

    lax.fori_loop(0, ZROWS, _zrow, 0)
    pltpu.sync_copy(zbuf, acc.at[pl.ds(s * ZROWS, ZROWS)])
    plsc.subcore_barrier()

    for t in range(STEPS):
        j = s + NSUB * t

        @pl.when(j < NFULL)
        def _():
            pltpu.make_async_copy(idx_hbm.at[pl.ds(j * CHUNK, CHUNK)],
                                  idx2.at[t], sem_i).wait()
            pltpu.make_async_copy(x_hbm.at[pl.ds(j * CHUNK, CHUNK)],
                                  xbuf.at[t], sem_x).wait()

        @pl.when(j == NFULL + 1)
        def _():
            pltpu.make_async_copy(idx_hbm.at[pl.ds(NFULL * CHUNK, TAIL)],
                                  idx2.at[STEPS - 1, pl.ds(0, TAIL)],
                                  sem_i).wait()
            pltpu.make_async_copy(x_hbm.at[pl.ds(NFULL * CHUNK, TAIL)],
                                  xbuf.at[STEPS - 1, pl.ds(0, TAIL)],
                                  sem_x).wait()

    for t in range(STEPS):
        j = s + NSUB * t

        @pl.when(j < NFULL)
        def _():
            pltpu.sync_copy(xbuf.at[t], acc.at[idx2.at[t]], add=True)

        @pl.when(j == NFULL + 1)
        def _():
            for q in range(TAIL // 16, CHUNK // 16):
                idx2[STEPS - 1, pl.ds(16 * q, 16)] = jnp.full(
                    (16,), TRASH, jnp.int32)
            pltpu.sync_copy(xbuf.at[STEPS - 1],
                            acc.at[idx2.at[STEPS - 1]], add=True)

    plsc.subcore_barrier()

    @pl.when(s < 15)
    def _():
        pltpu.sync_copy(acc.at[pl.ds(s * ZROWS, ZROWS)], zbuf)
        pltpu.sync_copy(zbuf, out_hbm.at[pl.ds(s * ZROWS, ZROWS)])

    @pl.when(s == 15)
    def _():
        pltpu.sync_copy(acc.at[pl.ds(15 * ZROWS, OTAIL)],
                        zbuf.at[pl.ds(0, OTAIL)])
        pltpu.sync_copy(zbuf.at[pl.ds(0, OTAIL)],
                        out_hbm.at[pl.ds(15 * ZROWS, OTAIL)])


@jax.jit
def _segment_sum_sc(x, index):
    mesh = plsc.VectorSubcoreMesh(core_axis_name="c", subcore_axis_name="s",
                                  num_cores=1)
    f = pl.kernel(
        _body,
        out_type=jax.ShapeDtypeStruct((K, D), jnp.float32),
        mesh=mesh,
        scratch_types=[
            pltpu.VMEM((STEPS, CHUNK), jnp.int32),
            pltpu.VMEM((STEPS, CHUNK, D), jnp.float32),
            pltpu.VMEM((ZROWS, D), jnp.float32),
            pltpu.VMEM_SHARED((ACC_ROWS, D), jnp.float32),
            pltpu.SemaphoreType.DMA,
            pltpu.SemaphoreType.DMA,
        ],
    )
    return f(x, index)


def kernel(x, adj, index, W, b):
    del adj, W, b
    return _segment_sum_sc(x, index)


# speedup vs baseline: 300.6583x; 1.0371x over previous
import jax
import jax.numpy as jnp
from jax import lax
from jax.experimental import pallas as pl
from jax.experimental.pallas import tpu as pltpu
from jax.experimental.pallas import tpu_sc as plsc

N = 10000
K = 2500
D = 128

CHUNK = 128
NFULL = N // CHUNK
TAIL = N - NFULL * CHUNK
NSUB = 16
STEPS = 5
ACC_ROWS = 2560
TRASH = ACC_ROWS - 1
ZROWS = ACC_ROWS // NSUB
OTAIL = K - 15 * ZROWS


def _body(x_hbm, idx_hbm, out_hbm, idx2, xbuf, zbuf, acc, sem_i, sem_x):
    s = lax.axis_index("s")

    for t in range(STEPS):
        j = s + NSUB * t

        @pl.when(j < NFULL)
        def _():
            pltpu.async_copy(idx_hbm.at[pl.ds(j * CHUNK, CHUNK)],
                             idx2.at[t], sem_i.at[t])
            pltpu.async_copy(x_hbm.at[pl.ds(j * CHUNK, CHUNK)],
                             xbuf.at[t], sem_x.at[t])

        @pl.when(j == NFULL + 1)
        def _():
            pltpu.async_copy(idx_hbm.at[pl.ds(NFULL * CHUNK, TAIL)],
                             idx2.at[STEPS - 1, pl.ds(0, TAIL)],
                             sem_i.at[STEPS - 1])
            pltpu.async_copy(x_hbm.at[pl.ds(NFULL * CHUNK, TAIL)],
                             xbuf.at[STEPS - 1, pl.ds(0, TAIL)],
                             sem_x.at[STEPS - 1])

    def _zrow(i, carry):
        for g in range(D // 16):
            zbuf[i, pl.ds(16 * g, 16)] = jnp.zeros((16,), jnp.float32)
        return carry

    lax.fori_loop(0, ZROWS, _zrow, 0)
    pltpu.sync_copy(zbuf, acc.at[pl.ds(s * ZROWS, ZROWS)])
    plsc.subcore_barrier()

    for t in range(STEPS):
        j = s + NSUB * t

        @pl.when(j < NFULL)
        def _():
            pltpu.make_async_copy(idx_hbm.at[pl.ds(j * CHUNK, CHUNK)],
                                  idx2.at[t], sem_i.at[t]).wait()
            pltpu.make_async_copy(x_hbm.at[pl.ds(j * CHUNK, CHUNK)],
                                  xbuf.at[t], sem_x.at[t]).wait()
            pltpu.sync_copy(xbuf.at[t], acc.at[idx2.at[t]], add=True)

        @pl.when(j == NFULL + 1)
        def _():
            pltpu.make_async_copy(idx_hbm.at[pl.ds(NFULL * CHUNK, TAIL)],
                                  idx2.at[STEPS - 1, pl.ds(0, TAIL)],
                                  sem_i.at[STEPS - 1]).wait()
            pltpu.make_async_copy(x_hbm.at[pl.ds(NFULL * CHUNK, TAIL)],
                                  xbuf.at[STEPS - 1, pl.ds(0, TAIL)],
                                  sem_x.at[STEPS - 1]).wait()
            for q in range(TAIL // 16, CHUNK // 16):
                idx2[STEPS - 1, pl.ds(16 * q, 16)] = jnp.full(
                    (16,), TRASH, jnp.int32)
            pltpu.sync_copy(xbuf.at[STEPS - 1],
                            acc.at[idx2.at[STEPS - 1]], add=True)

    plsc.subcore_barrier()

    @pl.when(s < 15)
    def _():
        pltpu.sync_copy(acc.at[pl.ds(s * ZROWS, ZROWS)],
                        out_hbm.at[pl.ds(s * ZROWS, ZROWS)])

    @pl.when(s == 15)
    def _():
        pltpu.sync_copy(acc.at[pl.ds(15 * ZROWS, OTAIL)],
                        out_hbm.at[pl.ds(15 * ZROWS, OTAIL)])


@jax.jit
def _segment_sum_sc(x, index):
    mesh = plsc.VectorSubcoreMesh(core_axis_name="c", subcore_axis_name="s",
                                  num_cores=1)
    f = pl.kernel(
        _body,
        out_type=jax.ShapeDtypeStruct((K, D), jnp.float32),
        mesh=mesh,
        scratch_types=[
            pltpu.VMEM((STEPS, CHUNK), jnp.int32),
            pltpu.VMEM((STEPS, CHUNK, D), jnp.float32),
            pltpu.VMEM((ZROWS, D), jnp.float32),
            pltpu.VMEM_SHARED((ACC_ROWS, D), jnp.float32),
            pltpu.SemaphoreType.DMA((STEPS,)),
            pltpu.SemaphoreType.DMA((STEPS,)),
        ],
    )
    return f(x, index)


def kernel(x, adj, index, W, b):
    del adj, W, b
    return _segment_sum_sc(x, index)
